# baseline (speedup 1.0000x reference)
import jax
import jax.numpy as jnp
import numpy as np
from jax.experimental import pallas as pl

_H = 2048
_W = 2048
_C = 3
_LANES = _W * _C

def _make_rnd3() -> np.ndarray:
    def draw():
        return jax.random.randint(jax.random.key(1), (_H, _W), 0, 255, dtype=jnp.int16)

    try:
        with jax.default_device(jax.devices("cpu")[0]):
            r = np.asarray(draw())
    except Exception:
        r = np.asarray(draw())
    return np.repeat(r.astype(np.uint8), _C, axis=1)


_RND3 = _make_rnd3()


def _body(x_ref, r_ref, o_ref):
    x = x_ref[...]
    c = jnp.where(x == 255.0, 4, 0) + jnp.where(x == 0.0, 1, 0)
    s = c + jnp.roll(c, -1, axis=1) + jnp.roll(c, 1, axis=1)
    j3 = jax.lax.broadcasted_iota(jnp.int32, x.shape, 1) % 3
    s_full = jnp.where(j3 == 1, s, jnp.where(j3 == 0, jnp.roll(s, -1, axis=1), jnp.roll(s, 1, axis=1)))
    line = s_full == 12
    back = s_full == 3
    rnd = r_ref[...].astype(jnp.int32)
    out = jnp.where(line, 0, jnp.where(back, rnd, x.astype(jnp.int32)))
    o_ref[...] = out.astype(jnp.uint8)


def kernel(input):
    x2d = input.reshape(_H, _LANES)
    rows = 128
    grid = (_H // rows,)
    out = pl.pallas_call(
        _body,
        grid=grid,
        in_specs=[
            pl.BlockSpec((rows, _LANES), lambda i: (i, 0)),
            pl.BlockSpec((rows, _LANES), lambda i: (i, 0)),
        ],
        out_specs=pl.BlockSpec((rows, _LANES), lambda i: (i, 0)),
        out_shape=jax.ShapeDtypeStruct((_H, _LANES), jnp.uint8),
    )(x2d, jnp.asarray(_RND3))
    return out.reshape(_H, _W, _C)


# speedup vs baseline: 12.0432x; 12.0432x over previous
import jax
import jax.numpy as jnp
import numpy as np
from jax.experimental import pallas as pl

_H = 2048
_W = 2048
_C = 3


def _make_rnd() -> np.ndarray:
    def draw():
        return jax.random.randint(jax.random.key(1), (_H, _W), 0, 255, dtype=jnp.int16)

    try:
        with jax.default_device(jax.devices("cpu")[0]):
            r = np.asarray(draw())
    except Exception:
        r = np.asarray(draw())
    return r.astype(np.uint8)


_RND = _make_rnd()


def _body(x_ref, r_ref, o_ref):
    x0 = x_ref[0]
    x1 = x_ref[1]
    x2 = x_ref[2]
    line = (x0 == 255.0) & (x1 == 255.0) & (x2 == 255.0)
    back = (x0 == 0.0) & (x1 == 0.0) & (x2 == 0.0)
    rnd = r_ref[...].astype(jnp.int32)
    for c, xc in enumerate((x0, x1, x2)):
        out = jnp.where(line, 0, jnp.where(back, rnd, xc.astype(jnp.int32)))
        o_ref[c] = out.astype(jnp.uint8)


def kernel(input):
    xp = jnp.transpose(input, (2, 0, 1))
    rows = 128
    grid = (_H // rows,)
    out = pl.pallas_call(
        _body,
        grid=grid,
        in_specs=[
            pl.BlockSpec((_C, rows, _W), lambda i: (0, i, 0)),
            pl.BlockSpec((rows, _W), lambda i: (i, 0)),
        ],
        out_specs=pl.BlockSpec((_C, rows, _W), lambda i: (0, i, 0)),
        out_shape=jax.ShapeDtypeStruct((_C, _H, _W), jnp.uint8),
    )(xp, jnp.asarray(_RND))
    return jnp.transpose(out, (1, 2, 0))


# speedup vs baseline: 13.5515x; 1.1252x over previous
import jax
import jax.numpy as jnp
import numpy as np
from jax.experimental import pallas as pl

_H = 2048
_W = 2048
_C = 3


def _make_rnd() -> np.ndarray:
    def draw():
        return jax.random.randint(jax.random.key(1), (_H, _W), 0, 255, dtype=jnp.int16)

    try:
        with jax.default_device(jax.devices("cpu")[0]):
            r = np.asarray(draw())
    except Exception:
        r = np.asarray(draw())
    return r.astype(np.uint8)


_RND = _make_rnd()


def _body(x_ref, r_ref, o_ref):
    x0 = x_ref[0]
    x1 = x_ref[1]
    x2 = x_ref[2]
    line = (x0 == 255.0) & (x1 == 255.0) & (x2 == 255.0)
    back = (x0 == 0.0) & (x1 == 0.0) & (x2 == 0.0)
    rnd = r_ref[...].astype(jnp.int32)
    for c, xc in enumerate((x0, x1, x2)):
        out = jnp.where(line, 0, jnp.where(back, rnd, xc.astype(jnp.int32)))
        o_ref[c] = out.astype(jnp.uint8)


def kernel(input):
    xp = jnp.transpose(input, (2, 0, 1))
    rows = 256
    grid = (_H // rows,)
    out = pl.pallas_call(
        _body,
        grid=grid,
        in_specs=[
            pl.BlockSpec((_C, rows, _W), lambda i: (0, i, 0)),
            pl.BlockSpec((rows, _W), lambda i: (i, 0)),
        ],
        out_specs=pl.BlockSpec((_C, rows, _W), lambda i: (0, i, 0)),
        out_shape=jax.ShapeDtypeStruct((_C, _H, _W), jnp.uint8),
    )(xp, jnp.asarray(_RND))
    return jnp.transpose(out, (1, 2, 0))


# speedup vs baseline: 13.5970x; 1.0034x over previous
import jax
import jax.numpy as jnp
import numpy as np
from jax.experimental import pallas as pl

_H = 2048
_W = 2048
_C = 3


def _make_rnd() -> np.ndarray:
    def draw():
        return jax.random.randint(jax.random.key(1), (_H, _W), 0, 255, dtype=jnp.int16)

    try:
        with jax.default_device(jax.devices("cpu")[0]):
            r = np.asarray(draw())
    except Exception:
        r = np.asarray(draw())
    return r.astype(np.uint8)


_RND = _make_rnd()


def _body(x_ref, r_ref, o_ref):
    x0 = x_ref[0]
    x1 = x_ref[1]
    x2 = x_ref[2]
    line = (x0 == 255.0) & (x1 == 255.0) & (x2 == 255.0)
    back = (x0 == 0.0) & (x1 == 0.0) & (x2 == 0.0)
    rnd = r_ref[...].astype(jnp.int32)
    for c, xc in enumerate((x0, x1, x2)):
        out = jnp.where(line, 0, jnp.where(back, rnd, xc.astype(jnp.int32)))
        o_ref[c] = out.astype(jnp.uint8)


def kernel(input):
    xp = jnp.transpose(input, (2, 0, 1))
    rows = 512
    grid = (_H // rows,)
    out = pl.pallas_call(
        _body,
        grid=grid,
        in_specs=[
            pl.BlockSpec((_C, rows, _W), lambda i: (0, i, 0)),
            pl.BlockSpec((rows, _W), lambda i: (i, 0)),
        ],
        out_specs=pl.BlockSpec((_C, rows, _W), lambda i: (0, i, 0)),
        out_shape=jax.ShapeDtypeStruct((_C, _H, _W), jnp.uint8),
    )(xp, jnp.asarray(_RND))
    return jnp.transpose(out, (1, 2, 0))
